# baseline (speedup 1.0000x reference)
import functools

import jax
import jax.numpy as jnp
from jax import lax
from jax.experimental import pallas as pl
from jax.experimental.pallas import tpu as pltpu
from jax.experimental.pallas import tpu_sc as plsc

N = 10000
NP = 10240
E = 320000
D = 128
G = 64
L = 4

NC = 2
NS = 16
NW = NC * NS
NPT = NP // NS
EPW = E // NW
EPT = E // NS
CH = 80
CHA = 40
NCHA = EPW // CHA

_f32 = jnp.float32


def _rsqrt16(x):
    i = lax.bitcast_convert_type(x, jnp.int32)
    i = jnp.int32(0x5F3759DF) - lax.shift_right_logical(i, 1)
    y = lax.bitcast_convert_type(i, _f32)
    for _ in range(3):
        y = y * (1.5 - 0.5 * x * y * y)
    return y


def _prep_body(row_hbm, deg_hbm, dis_hbm, disrow_hbm,
               idx0, idx1, idx2, idx3, idx4, ones_v, nslice_v, dslice_v,
               dis_v, rowe_v, dre_v, shared,
               si0, si1, si2, si3, si4, so0, so1, so2, so3, so4, sem):
    c = lax.axis_index("c")
    s = lax.axis_index("s")
    wid = c * NS + s
    idx = [idx0, idx1, idx2, idx3, idx4]
    si = [si0, si1, si2, si3, si4]
    so = [so0, so1, so2, so3, so4]
    nch = EPT // CH

    def zb(k, _):
        dslice_v[pl.ds(k * 16, 16)] = jnp.zeros((16,), _f32)
        return 0
    lax.fori_loop(0, NPT // 16, zb, 0)
    pltpu.sync_copy(dslice_v, shared.at[pl.ds(s * NPT, NPT)])
    for k in range(CH // 16):
        ones_v[pl.ds(k * 16, 16)] = jnp.ones((16,), _f32)
    plsc.subcore_barrier()

    def _start_load(k, j):
        pltpu.async_copy(row_hbm.at[pl.ds(s * EPT + k * CH, CH)],
                         idx[j], si[j])

    for j in range(4):
        _start_load(j, j)

    def deg_chunk(m, _):
        for j in range(5):
            k = 5 * m + j
            j1 = (j + 4) % 5
            pltpu.make_async_copy(row_hbm.at[pl.ds(0, CH)], idx[j],
                                  si[j]).wait()
            pltpu.async_copy(ones_v, shared.at[idx[j]], so[j], add=True)

            @pl.when(k >= 1)
            def _():
                pltpu.make_async_copy(ones_v, shared.at[pl.ds(0, CH)],
                                      so[j1]).wait()

            @pl.when(k + 4 < nch)
            def _():
                _start_load(k + 4, j1)
        return 0
    lax.fori_loop(0, nch // 5, deg_chunk, 0)
    pltpu.make_async_copy(ones_v, shared.at[pl.ds(0, CH)], so[4]).wait()
    plsc.subcore_barrier()

    pltpu.sync_copy(shared.at[pl.ds(s * NPT, NPT)], nslice_v)

    def rs(k, _):
        dg = nslice_v[pl.ds(k * 16, 16)] + 1.0
        nslice_v[pl.ds(k * 16, 16)] = dg
        dslice_v[pl.ds(k * 16, 16)] = _rsqrt16(dg)
        return 0
    lax.fori_loop(0, NPT // 16, rs, 0)

    @pl.when(c == 0)
    def _():
        pltpu.sync_copy(nslice_v, deg_hbm.at[pl.ds(s * NPT, NPT)])
        pltpu.sync_copy(dslice_v, dis_hbm.at[pl.ds(s * NPT, NPT)])

    pltpu.sync_copy(dslice_v, shared.at[pl.ds(s * NPT, NPT)])
    plsc.subcore_barrier()
    pltpu.sync_copy(shared, dis_v)
    pltpu.sync_copy(row_hbm.at[pl.ds(wid * EPW, EPW)], rowe_v)

    def gth(k, _):
        idx16 = rowe_v[pl.ds(k * 16, 16)]
        dre_v[pl.ds(k * 16, 16)] = plsc.load_gather(dis_v, [idx16])
        return 0
    lax.fori_loop(0, EPW // 16, gth, 0)
    pltpu.sync_copy(dre_v, disrow_hbm.at[pl.ds(wid * EPW, EPW)])


def _prep_call(row):
    return pl.kernel(
        _prep_body,
        out_type=[jax.ShapeDtypeStruct((NP,), _f32),
                  jax.ShapeDtypeStruct((NP,), _f32),
                  jax.ShapeDtypeStruct((E,), _f32)],
        mesh=plsc.VectorSubcoreMesh(core_axis_name="c", subcore_axis_name="s"),
        compiler_params=pltpu.CompilerParams(needs_layout_passes=False),
        scratch_types=(
            [pltpu.VMEM((CH,), jnp.int32)] * 5
            + [pltpu.VMEM((CH,), _f32),
               pltpu.VMEM((NPT,), _f32),
               pltpu.VMEM((NPT,), _f32),
               pltpu.VMEM((NP,), _f32),
               pltpu.VMEM((EPW,), jnp.int32),
               pltpu.VMEM((EPW,), _f32),
               pltpu.VMEM_SHARED((NP,), _f32)]
            + [pltpu.SemaphoreType.DMA] * 11
        ),
    )(row)


def _agg_body(layer, hl2_hbm, ee_hbm, row_hbm, col_hbm,
              agg0_hbm, agg1_hbm,
              rowall, c0, c1, g0, g1, e0, e1, o0, o1, shared,
              sg0, sg1, se0, se1, sc0, sc1, sc2, sc3, so0, so1):
    c = lax.axis_index("c")
    s = lax.axis_index("s")
    wid = c * NS + s
    colv = [c0, c1]
    g = [g0, g1]
    ev = [e0, e1]
    ov = [o0, o1]
    sg = [sg0, sg1]
    se = [se0, se1]
    sc = [sc0, sc1, sc2, sc3]
    so = [so0, so1]
    ebase = wid * EPW

    def zg(i, _):
        for j in range(D // 16):
            g0[i, pl.ds(j * 16, 16)] = jnp.zeros((16,), _f32)
        return 0
    lax.fori_loop(0, CHA, zg, 0)
    nz = NPT // CHA
    for m in range(nz):
        j = m % 4
        if m >= 4:
            pltpu.make_async_copy(g0, shared.at[pl.ds(0, CHA)], sc[j]).wait()
        pltpu.async_copy(g0, shared.at[pl.ds(s * NPT + m * CHA, CHA)], sc[j])
    for j in range(4):
        pltpu.make_async_copy(g0, shared.at[pl.ds(0, CHA)], sc[j]).wait()

    pltpu.async_copy(row_hbm.at[pl.ds(ebase, EPW)], rowall, sg0).wait()
    plsc.subcore_barrier()

    def _start_in(k, b):
        pltpu.async_copy(
            hl2_hbm.at[rowall.at[pl.ds(k * CHA, CHA)]], g[b], sg[b])
        pltpu.async_copy(
            ee_hbm.at[layer, pl.ds((ebase + k * CHA) // 2, CHA // 2)],
            ev[b], se[b])

    _start_in(0, 0)
    _start_in(1, 1)

    def chunk(k2, _):
        for b in range(2):
            k = 2 * k2 + b
            pltpu.make_async_copy(hl2_hbm.at[pl.ds(0, CHA)], g[b],
                                  sg[b]).wait()
            pltpu.make_async_copy(ee_hbm.at[0, pl.ds(0, CHA // 2)], ev[b],
                                  se[b]).wait()

            @pl.when(k2 >= 1)
            def _():
                pltpu.make_async_copy(ov[b], shared.at[pl.ds(0, CHA)],
                                      so[b]).wait()
            pltpu.async_copy(col_hbm.at[pl.ds(ebase + k * CHA, CHA)],
                             colv[b], sc[b])

            def edge(pi, _):
                i0 = 2 * pi
                i1 = 2 * pi + 1
                for jj in range(D // 16):
                    w16 = ev[b][pi, pl.ds(jj * 16, 16)]
                    bf = plsc.bitcast(w16, jnp.bfloat16)
                    elo, ehi = plsc.unpack(
                        bf, format=plsc.PackFormat.INTERLEAVED)
                    a0 = g[b][i0, pl.ds(jj * 16, 16)]
                    a1 = g[b][i1, pl.ds(jj * 16, 16)]
                    ov[b][i0, pl.ds(jj * 16, 16)] = jnp.maximum(
                        a0 + elo, 0.0)
                    ov[b][i1, pl.ds(jj * 16, 16)] = jnp.maximum(
                        a1 + ehi, 0.0)
                return 0
            lax.fori_loop(0, CHA // 2, edge, 0)

            @pl.when(k2 < NCHA // 2 - 1)
            def _():
                _start_in(k + 2, b)
            pltpu.make_async_copy(col_hbm.at[pl.ds(0, CHA)], colv[b],
                                  sc[b]).wait()
            pltpu.async_copy(ov[b], shared.at[colv[b]], so[b], add=True)
        return 0
    lax.fori_loop(0, NCHA // 2, chunk, 0)
    for b in range(2):
        pltpu.make_async_copy(ov[b], shared.at[pl.ds(0, CHA)], so[b]).wait()
    plsc.subcore_barrier()

    bufs = [g0, g1, o0, o1]
    sa = [sg0, sg1, se0, se1]
    nd = NPT // CHA
    for m in range(nd):
        j = m % 4
        if m >= 4:
            pltpu.make_async_copy(bufs[j], agg0_hbm.at[pl.ds(0, CHA)],
                                  sc[j]).wait()
        pltpu.async_copy(shared.at[pl.ds(s * NPT + m * CHA, CHA)],
                         bufs[j], sa[j]).wait()

        @pl.when(c == 0)
        def _():
            pltpu.async_copy(bufs[j],
                             agg0_hbm.at[pl.ds(s * NPT + m * CHA, CHA)],
                             sc[j])

        @pl.when(c == 1)
        def _():
            pltpu.async_copy(bufs[j],
                             agg1_hbm.at[pl.ds(s * NPT + m * CHA, CHA)],
                             sc[j])
    for j in range(4):
        pltpu.make_async_copy(bufs[j], agg0_hbm.at[pl.ds(0, CHA)],
                              sc[j]).wait()


def _agg_call(layer, hl2, ee2, row, col):
    return pl.kernel(
        functools.partial(_agg_body, layer),
        out_type=[jax.ShapeDtypeStruct((NP, D), _f32),
                  jax.ShapeDtypeStruct((NP, D), _f32)],
        mesh=plsc.VectorSubcoreMesh(core_axis_name="c", subcore_axis_name="s"),
        compiler_params=pltpu.CompilerParams(needs_layout_passes=False,
                                             use_tc_tiling_on_sc=False),
        scratch_types=(
            [pltpu.VMEM((EPW,), jnp.int32),
             pltpu.VMEM((CHA,), jnp.int32),
             pltpu.VMEM((CHA,), jnp.int32)]
            + [pltpu.VMEM((CHA, D), _f32)] * 2
            + [pltpu.VMEM((CHA // 2, D), jnp.int32)] * 2
            + [pltpu.VMEM((CHA, D), _f32)] * 2
            + [pltpu.VMEM_SHARED((NP, D), _f32)]
            + [pltpu.SemaphoreType.DMA] * 10
        ),
    )(hl2, ee2, row, col)


def _ee_body(be, ea_ref, w_ref, o_ref):
    ee = jnp.dot(ea_ref[...], w_ref[0], preferred_element_type=_f32)
    bits = lax.bitcast_convert_type(ee, jnp.uint32)
    rnd = bits + jnp.uint32(0x7FFF) + (
        lax.shift_right_logical(bits, jnp.uint32(16)) & jnp.uint32(1))
    b16 = lax.shift_right_logical(rnd, jnp.uint32(16))
    b16 = b16.reshape(be // 2, 2, D)
    packed = b16[:, 0, :] | lax.shift_left(b16[:, 1, :], jnp.uint32(16))
    o_ref[0] = lax.bitcast_convert_type(packed, jnp.int32)


def _ee_call(ea_pad, w_all):
    be = 4000
    return pl.pallas_call(
        functools.partial(_ee_body, be),
        grid=(L, E // be),
        in_specs=[
            pl.BlockSpec((be, 8), lambda l, e: (e, 0)),
            pl.BlockSpec((1, 8, D), lambda l, e: (l, 0, 0)),
        ],
        out_specs=pl.BlockSpec((1, be // 2, D), lambda l, e: (l, e, 0)),
        out_shape=jax.ShapeDtypeStruct((L, E // 2, D), jnp.int32),
    )(ea_pad, w_all)


def _node0_body(emb_ref, dis_ref, w_ref, b_ref, h_ref, hl_ref, hl2_ref):
    h = jnp.broadcast_to(emb_ref[...], (N, D))
    hl = jnp.dot(h, w_ref[...], preferred_element_type=_f32) + b_ref[...]
    h_ref[...] = h
    hl_ref[...] = hl
    hl2_ref[0:N] = dis_ref[...] * hl
    hl2_ref[N:NP] = jnp.zeros((NP - N, D), _f32)


def _node0_call(emb, dis, w0t, b0):
    return pl.pallas_call(
        _node0_body,
        out_shape=[jax.ShapeDtypeStruct((N, D), _f32),
                   jax.ShapeDtypeStruct((N, D), _f32),
                   jax.ShapeDtypeStruct((NP, D), _f32)],
    )(emb, dis, w0t, b0)


def _bn_update(agg0, agg1, hl_prev, h_prev, deg, dis, root, gamma, beta,
               do_relu):
    agg = dis * (agg0[0:N] + agg1[0:N])
    out = agg + jnp.maximum(hl_prev + root, 0.0) / deg
    mu = jnp.mean(out, axis=0, keepdims=True)
    var = jnp.mean((out - mu) ** 2, axis=0, keepdims=True)
    out = (out - mu) * jax.lax.rsqrt(var + 1e-5) * gamma + beta
    if do_relu:
        out = jnp.maximum(out, 0.0)
    return out + h_prev


def _node_body(do_relu, agg0_ref, agg1_ref, hlp_ref, hp_ref, deg_ref, dis_ref,
               root_ref, gamma_ref, beta_ref, w_ref, b_ref,
               h_ref, hl_ref, hl2_ref):
    h = _bn_update(agg0_ref[...], agg1_ref[...], hlp_ref[...], hp_ref[...],
                   deg_ref[...], dis_ref[...], root_ref[...], gamma_ref[...],
                   beta_ref[...], do_relu)
    hl = jnp.dot(h, w_ref[...], preferred_element_type=_f32) + b_ref[...]
    h_ref[...] = h
    hl_ref[...] = hl
    hl2_ref[0:N] = dis_ref[...] * hl
    hl2_ref[N:NP] = jnp.zeros((NP - N, D), _f32)


def _node_call(do_relu, agg0, agg1, hl_prev, h_prev, deg, dis, root, gamma,
               beta, wt, b):
    return pl.pallas_call(
        functools.partial(_node_body, do_relu),
        out_shape=[jax.ShapeDtypeStruct((N, D), _f32),
                   jax.ShapeDtypeStruct((N, D), _f32),
                   jax.ShapeDtypeStruct((NP, D), _f32)],
    )(agg0, agg1, hl_prev, h_prev, deg, dis, root, gamma, beta, wt, b)


def _final_body(agg0_ref, agg1_ref, hlp_ref, hp_ref, deg_ref, dis_ref,
                root_ref, gamma_ref, beta_ref, batch_ref, wp_ref, bp_ref,
                o_ref):
    h = _bn_update(agg0_ref[...], agg1_ref[...], hlp_ref[...], hp_ref[...],
                   deg_ref[...], dis_ref[...], root_ref[...], gamma_ref[...],
                   beta_ref[...], do_relu=False)
    gids = lax.broadcasted_iota(jnp.int32, (1, G), 1)
    m = (batch_ref[...] == gids).astype(_f32)
    hg_sum = lax.dot_general(m, h, (((0,), (0,)), ((), ())),
                             preferred_element_type=_f32)
    counts = lax.dot_general(m, jnp.ones((N, 1), _f32),
                             (((0,), (0,)), ((), ())),
                             preferred_element_type=_f32)
    hg = hg_sum / jnp.maximum(counts, 1.0)
    o_ref[...] = jnp.dot(hg, wp_ref[...], preferred_element_type=_f32) \
        + bp_ref[...]


def _final_call(agg0, agg1, hl_prev, h_prev, deg, dis, root, gamma, beta,
                batch, wpt, bp):
    out_dim = wpt.shape[1]
    return pl.pallas_call(
        _final_body,
        out_shape=jax.ShapeDtypeStruct((G, out_dim), _f32),
    )(agg0, agg1, hl_prev, h_prev, deg, dis, root, gamma, beta, batch, wpt, bp)


def kernel(x, edge_index, edge_attr, batch, params):
    row = edge_index[0]
    col = edge_index[1]

    deg_p, dis_p, dis_row = _prep_call(row)
    deg = deg_p[:N].reshape(N, 1)
    dis = dis_p[:N].reshape(N, 1)

    ea_pad = jnp.concatenate([edge_attr, jnp.ones((E, 1), _f32)], axis=1)
    ea_pad = ea_pad * dis_row[:, None]
    w_all = jnp.concatenate(
        [jnp.transpose(params['W_edge'], (0, 2, 1)),
         params['b_edge'][:, None, :]], axis=1)
    ee2 = _ee_call(ea_pad, w_all)

    h, hl, hl2 = _node0_call(params['node_emb'], dis,
                             params['W_lin'][0].T, params['b_lin'][0][None])

    for l in range(1, L):
        agg0, agg1 = _agg_call(l - 1, hl2, ee2, row, col)
        h, hl, hl2 = _node_call(
            l - 1 < L - 1, agg0, agg1, hl, h, deg, dis,
            params['root'][l - 1], params['gamma'][l - 1][None],
            params['beta'][l - 1][None],
            params['W_lin'][l].T, params['b_lin'][l][None])

    agg0, agg1 = _agg_call(L - 1, hl2, ee2, row, col)
    return _final_call(agg0, agg1, hl, h, deg, dis,
                       params['root'][L - 1], params['gamma'][L - 1][None],
                       params['beta'][L - 1][None],
                       batch.reshape(N, 1), params['W_pred'].T,
                       params['b_pred'][None])


# speedup vs baseline: 1.2065x; 1.2065x over previous
import functools

import jax
import jax.numpy as jnp
from jax import lax
from jax.experimental import pallas as pl
from jax.experimental.pallas import tpu as pltpu
from jax.experimental.pallas import tpu_sc as plsc

N = 10000
NP = 10240
E = 320000
D = 128
G = 64
L = 4

NC = 2
NS = 16
NW = NC * NS
NPT = NP // NS
EPW = E // NW
EPT = E // NS
CH = 80
CHA = 40
NCHA = EPW // CHA

_f32 = jnp.float32


def _rsqrt16(x):
    i = lax.bitcast_convert_type(x, jnp.int32)
    i = jnp.int32(0x5F3759DF) - lax.shift_right_logical(i, 1)
    y = lax.bitcast_convert_type(i, _f32)
    for _ in range(3):
        y = y * (1.5 - 0.5 * x * y * y)
    return y


def _prep_body(row_hbm, deg_hbm, dis_hbm, disrow_hbm,
               idx0, idx1, idx2, idx3, idx4, ones_v, nslice_v, dslice_v,
               dis_v, rowe_v, dre_v, shared,
               si0, si1, si2, si3, si4, so0, so1, so2, so3, so4, sem):
    c = lax.axis_index("c")
    s = lax.axis_index("s")
    wid = c * NS + s
    idx = [idx0, idx1, idx2, idx3, idx4]
    si = [si0, si1, si2, si3, si4]
    so = [so0, so1, so2, so3, so4]
    nch = EPT // CH

    def zb(k, _):
        dslice_v[pl.ds(k * 16, 16)] = jnp.zeros((16,), _f32)
        return 0
    lax.fori_loop(0, NPT // 16, zb, 0)
    pltpu.sync_copy(dslice_v, shared.at[pl.ds(s * NPT, NPT)])
    for k in range(CH // 16):
        ones_v[pl.ds(k * 16, 16)] = jnp.ones((16,), _f32)
    plsc.subcore_barrier()

    def _start_load(k, j):
        pltpu.async_copy(row_hbm.at[pl.ds(s * EPT + k * CH, CH)],
                         idx[j], si[j])

    for j in range(4):
        _start_load(j, j)

    def deg_chunk(m, _):
        for j in range(5):
            k = 5 * m + j
            j1 = (j + 4) % 5
            pltpu.make_async_copy(row_hbm.at[pl.ds(0, CH)], idx[j],
                                  si[j]).wait()
            pltpu.async_copy(ones_v, shared.at[idx[j]], so[j], add=True)

            @pl.when(k >= 1)
            def _():
                pltpu.make_async_copy(ones_v, shared.at[pl.ds(0, CH)],
                                      so[j1]).wait()

            @pl.when(k + 4 < nch)
            def _():
                _start_load(k + 4, j1)
        return 0
    lax.fori_loop(0, nch // 5, deg_chunk, 0)
    pltpu.make_async_copy(ones_v, shared.at[pl.ds(0, CH)], so[4]).wait()
    plsc.subcore_barrier()

    pltpu.sync_copy(shared.at[pl.ds(s * NPT, NPT)], nslice_v)

    def rs(k, _):
        dg = nslice_v[pl.ds(k * 16, 16)] + 1.0
        nslice_v[pl.ds(k * 16, 16)] = dg
        dslice_v[pl.ds(k * 16, 16)] = _rsqrt16(dg)
        return 0
    lax.fori_loop(0, NPT // 16, rs, 0)

    @pl.when(c == 0)
    def _():
        pltpu.sync_copy(nslice_v, deg_hbm.at[pl.ds(s * NPT, NPT)])
        pltpu.sync_copy(dslice_v, dis_hbm.at[pl.ds(s * NPT, NPT)])

    pltpu.sync_copy(dslice_v, shared.at[pl.ds(s * NPT, NPT)])
    plsc.subcore_barrier()
    pltpu.sync_copy(shared, dis_v)
    pltpu.sync_copy(row_hbm.at[pl.ds(wid * EPW, EPW)], rowe_v)

    def gth(k, _):
        idx16 = rowe_v[pl.ds(k * 16, 16)]
        dre_v[pl.ds(k * 16, 16)] = plsc.load_gather(dis_v, [idx16])
        return 0
    lax.fori_loop(0, EPW // 16, gth, 0)
    pltpu.sync_copy(dre_v, disrow_hbm.at[pl.ds(wid * EPW, EPW)])


def _prep_call(row):
    return pl.kernel(
        _prep_body,
        out_type=[jax.ShapeDtypeStruct((NP,), _f32),
                  jax.ShapeDtypeStruct((NP,), _f32),
                  jax.ShapeDtypeStruct((E,), _f32)],
        mesh=plsc.VectorSubcoreMesh(core_axis_name="c", subcore_axis_name="s"),
        compiler_params=pltpu.CompilerParams(needs_layout_passes=False),
        scratch_types=(
            [pltpu.VMEM((CH,), jnp.int32)] * 5
            + [pltpu.VMEM((CH,), _f32),
               pltpu.VMEM((NPT,), _f32),
               pltpu.VMEM((NPT,), _f32),
               pltpu.VMEM((NP,), _f32),
               pltpu.VMEM((EPW,), jnp.int32),
               pltpu.VMEM((EPW,), _f32),
               pltpu.VMEM_SHARED((NP,), _f32)]
            + [pltpu.SemaphoreType.DMA] * 11
        ),
    )(row)


def _agg_body(layer, with_gather, hl2_hbm, ee_hbm, row_hbm, col_hbm,
              agg0_hbm, agg1_hbm,
              rowall, c0, c1, g0, g1, e0, e1, o0, o1, shared,
              sg0, sg1, se0, se1, sc0, sc1, sc2, sc3, so0, so1):
    c = lax.axis_index("c")
    s = lax.axis_index("s")
    wid = c * NS + s
    colv = [c0, c1]
    g = [g0, g1]
    ev = [e0, e1]
    ov = [o0, o1]
    sg = [sg0, sg1]
    se = [se0, se1]
    sc = [sc0, sc1, sc2, sc3]
    so = [so0, so1]
    ebase = wid * EPW

    def zg(i, _):
        for j in range(D // 16):
            g0[i, pl.ds(j * 16, 16)] = jnp.zeros((16,), _f32)
        return 0
    lax.fori_loop(0, CHA, zg, 0)
    nz = NPT // CHA
    for m in range(nz):
        j = m % 4
        if m >= 4:
            pltpu.make_async_copy(g0, shared.at[pl.ds(0, CHA)], sc[j]).wait()
        pltpu.async_copy(g0, shared.at[pl.ds(s * NPT + m * CHA, CHA)], sc[j])
    for j in range(4):
        pltpu.make_async_copy(g0, shared.at[pl.ds(0, CHA)], sc[j]).wait()

    pltpu.async_copy(row_hbm.at[pl.ds(ebase, EPW)], rowall, sg0).wait()
    plsc.subcore_barrier()

    def _start_in(k, b):
        if with_gather:
            pltpu.async_copy(
                hl2_hbm.at[rowall.at[pl.ds(k * CHA, CHA)]], g[b], sg[b])
        pltpu.async_copy(
            ee_hbm.at[layer, pl.ds(ebase + k * CHA, CHA)], ev[b], se[b])

    _start_in(0, 0)
    _start_in(1, 1)

    def chunk(k2, _):
        for b in range(2):
            k = 2 * k2 + b
            if with_gather:
                pltpu.make_async_copy(hl2_hbm.at[pl.ds(0, CHA)], g[b],
                                      sg[b]).wait()
            pltpu.make_async_copy(hl2_hbm.at[pl.ds(0, CHA)], ev[b],
                                  se[b]).wait()

            @pl.when(k2 >= 1)
            def _():
                pltpu.make_async_copy(ov[b], shared.at[pl.ds(0, CHA)],
                                      so[b]).wait()
            pltpu.async_copy(col_hbm.at[pl.ds(ebase + k * CHA, CHA)],
                             colv[b], sc[b])

            def edge(i, _):
                for jj in range(D // 16):
                    bb = ev[b][i, pl.ds(jj * 16, 16)]
                    if with_gather:
                        bb = g[b][i, pl.ds(jj * 16, 16)] + bb
                    ov[b][i, pl.ds(jj * 16, 16)] = jnp.maximum(bb, 0.0)
                return 0
            lax.fori_loop(0, CHA, edge, 0)

            @pl.when(k2 < NCHA // 2 - 1)
            def _():
                _start_in(k + 2, b)
            pltpu.make_async_copy(col_hbm.at[pl.ds(0, CHA)], colv[b],
                                  sc[b]).wait()
            pltpu.async_copy(ov[b], shared.at[colv[b]], so[b], add=True)
        return 0
    lax.fori_loop(0, NCHA // 2, chunk, 0)
    for b in range(2):
        pltpu.make_async_copy(ov[b], shared.at[pl.ds(0, CHA)], so[b]).wait()
    plsc.subcore_barrier()

    bufs = [g0, g1, o0, o1]
    sa = [sg0, sg1, se0, se1]
    nd = NPT // CHA
    for m in range(nd):
        j = m % 4
        if m >= 4:
            pltpu.make_async_copy(bufs[j], agg0_hbm.at[pl.ds(0, CHA)],
                                  sc[j]).wait()
        pltpu.async_copy(shared.at[pl.ds(s * NPT + m * CHA, CHA)],
                         bufs[j], sa[j]).wait()

        @pl.when(c == 0)
        def _():
            pltpu.async_copy(bufs[j],
                             agg0_hbm.at[pl.ds(s * NPT + m * CHA, CHA)],
                             sc[j])

        @pl.when(c == 1)
        def _():
            pltpu.async_copy(bufs[j],
                             agg1_hbm.at[pl.ds(s * NPT + m * CHA, CHA)],
                             sc[j])
    for j in range(4):
        pltpu.make_async_copy(bufs[j], agg0_hbm.at[pl.ds(0, CHA)],
                              sc[j]).wait()


def _agg_call(layer, with_gather, hl2, ee2, row, col):
    return pl.kernel(
        functools.partial(_agg_body, layer, with_gather),
        out_type=[jax.ShapeDtypeStruct((NP, D), _f32),
                  jax.ShapeDtypeStruct((NP, D), _f32)],
        mesh=plsc.VectorSubcoreMesh(core_axis_name="c", subcore_axis_name="s"),
        compiler_params=pltpu.CompilerParams(needs_layout_passes=False),
        scratch_types=(
            [pltpu.VMEM((EPW,), jnp.int32),
             pltpu.VMEM((CHA,), jnp.int32),
             pltpu.VMEM((CHA,), jnp.int32)]
            + [pltpu.VMEM((CHA, D), _f32)] * 6
            + [pltpu.VMEM_SHARED((NP, D), _f32)]
            + [pltpu.SemaphoreType.DMA] * 10
        ),
    )(hl2, ee2, row, col)


def _ee_body(be, ea_ref, w_ref, o_ref):
    ea2c = ea_ref[...]
    w = w_ref[0]
    parts = [jnp.dot(ea2c[:, 8 * u:8 * u + 8], w,
                     preferred_element_type=_f32) for u in range(16)]
    o_ref[0] = jnp.stack(parts, axis=1).reshape(be, D)


def _ee_call(ea2c, w_all):
    be = 3200
    return pl.pallas_call(
        functools.partial(_ee_body, be),
        grid=(L, E // be),
        in_specs=[
            pl.BlockSpec((be // 16, 128), lambda l, e: (e, 0)),
            pl.BlockSpec((1, 8, D), lambda l, e: (l, 0, 0)),
        ],
        out_specs=pl.BlockSpec((1, be, D), lambda l, e: (l, e, 0)),
        out_shape=jax.ShapeDtypeStruct((L, E, D), _f32),
    )(ea2c, w_all)


def _node0_body(emb_ref, dis_ref, w_ref, b_ref, h_ref, hl_ref, hl2_ref):
    h = jnp.broadcast_to(emb_ref[...], (N, D))
    hl = jnp.dot(h, w_ref[...], preferred_element_type=_f32) + b_ref[...]
    h_ref[...] = h
    hl_ref[...] = hl
    hl2_ref[0:N] = dis_ref[...] * hl
    hl2_ref[N:NP] = jnp.zeros((NP - N, D), _f32)


def _node0_call(emb, dis, w0t, b0):
    return pl.pallas_call(
        _node0_body,
        out_shape=[jax.ShapeDtypeStruct((N, D), _f32),
                   jax.ShapeDtypeStruct((N, D), _f32),
                   jax.ShapeDtypeStruct((NP, D), _f32)],
    )(emb, dis, w0t, b0)


def _bn_update(agg0, agg1, hl_prev, h_prev, deg, dis, root, gamma, beta,
               do_relu):
    agg = dis * (agg0[0:N] + agg1[0:N])
    out = agg + jnp.maximum(hl_prev + root, 0.0) / deg
    mu = jnp.mean(out, axis=0, keepdims=True)
    var = jnp.mean((out - mu) ** 2, axis=0, keepdims=True)
    out = (out - mu) * jax.lax.rsqrt(var + 1e-5) * gamma + beta
    if do_relu:
        out = jnp.maximum(out, 0.0)
    return out + h_prev


def _node_body(do_relu, agg0_ref, agg1_ref, hlp_ref, hp_ref, deg_ref, dis_ref,
               root_ref, gamma_ref, beta_ref, w_ref, b_ref,
               h_ref, hl_ref, hl2_ref):
    h = _bn_update(agg0_ref[...], agg1_ref[...], hlp_ref[...], hp_ref[...],
                   deg_ref[...], dis_ref[...], root_ref[...], gamma_ref[...],
                   beta_ref[...], do_relu)
    hl = jnp.dot(h, w_ref[...], preferred_element_type=_f32) + b_ref[...]
    h_ref[...] = h
    hl_ref[...] = hl
    hl2_ref[0:N] = dis_ref[...] * hl
    hl2_ref[N:NP] = jnp.zeros((NP - N, D), _f32)


def _node_call(do_relu, agg0, agg1, hl_prev, h_prev, deg, dis, root, gamma,
               beta, wt, b):
    return pl.pallas_call(
        functools.partial(_node_body, do_relu),
        out_shape=[jax.ShapeDtypeStruct((N, D), _f32),
                   jax.ShapeDtypeStruct((N, D), _f32),
                   jax.ShapeDtypeStruct((NP, D), _f32)],
    )(agg0, agg1, hl_prev, h_prev, deg, dis, root, gamma, beta, wt, b)


def _final_body(agg0_ref, agg1_ref, hlp_ref, hp_ref, deg_ref, dis_ref,
                root_ref, gamma_ref, beta_ref, batch_ref, wp_ref, bp_ref,
                o_ref):
    h = _bn_update(agg0_ref[...], agg1_ref[...], hlp_ref[...], hp_ref[...],
                   deg_ref[...], dis_ref[...], root_ref[...], gamma_ref[...],
                   beta_ref[...], do_relu=False)
    gids = lax.broadcasted_iota(jnp.int32, (1, G), 1)
    m = (batch_ref[...] == gids).astype(_f32)
    hg_sum = lax.dot_general(m, h, (((0,), (0,)), ((), ())),
                             preferred_element_type=_f32)
    counts = lax.dot_general(m, jnp.ones((N, 1), _f32),
                             (((0,), (0,)), ((), ())),
                             preferred_element_type=_f32)
    hg = hg_sum / jnp.maximum(counts, 1.0)
    o_ref[...] = jnp.dot(hg, wp_ref[...], preferred_element_type=_f32) \
        + bp_ref[...]


def _final_call(agg0, agg1, hl_prev, h_prev, deg, dis, root, gamma, beta,
                batch, wpt, bp):
    out_dim = wpt.shape[1]
    return pl.pallas_call(
        _final_body,
        out_shape=jax.ShapeDtypeStruct((G, out_dim), _f32),
    )(agg0, agg1, hl_prev, h_prev, deg, dis, root, gamma, beta, batch, wpt, bp)


def kernel(x, edge_index, edge_attr, batch, params):
    row = edge_index[0]
    col = edge_index[1]

    deg_p, dis_p, dis_row = _prep_call(row)
    deg = deg_p[:N].reshape(N, 1)
    dis = dis_p[:N].reshape(N, 1)

    hl0row = params['node_emb'][0] @ params['W_lin'][0].T + params['b_lin'][0]
    ea_pad = jnp.concatenate([edge_attr, jnp.ones((E, 1), _f32)], axis=1)
    ea2c = (ea_pad * dis_row[:, None]).reshape(E // 16, 128)
    bias = params['b_edge'].at[0].add(hl0row)
    w_all = jnp.concatenate(
        [jnp.transpose(params['W_edge'], (0, 2, 1)),
         bias[:, None, :]], axis=1)
    ee2 = _ee_call(ea2c, w_all)

    h, hl, hl2 = _node0_call(params['node_emb'], dis,
                             params['W_lin'][0].T, params['b_lin'][0][None])

    for l in range(1, L):
        agg0, agg1 = _agg_call(l - 1, l - 1 > 0, hl2, ee2, row, col)
        h, hl, hl2 = _node_call(
            l - 1 < L - 1, agg0, agg1, hl, h, deg, dis,
            params['root'][l - 1], params['gamma'][l - 1][None],
            params['beta'][l - 1][None],
            params['W_lin'][l].T, params['b_lin'][l][None])

    agg0, agg1 = _agg_call(L - 1, True, hl2, ee2, row, col)
    return _final_call(agg0, agg1, hl, h, deg, dis,
                       params['root'][L - 1], params['gamma'][L - 1][None],
                       params['beta'][L - 1][None],
                       batch.reshape(N, 1), params['W_pred'].T,
                       params['b_pred'][None])


# speedup vs baseline: 1.3661x; 1.1323x over previous
import functools

import jax
import jax.numpy as jnp
from jax import lax
from jax.experimental import pallas as pl
from jax.experimental.pallas import tpu as pltpu
from jax.experimental.pallas import tpu_sc as plsc

N = 10000
NP = 10240
E = 320000
D = 128
G = 64
L = 4

NC = 2
NS = 16
NW = NC * NS
NPT = NP // NS
EPW = E // NW
EPT = E // NS
CH = 80
CHA = 40
NCHA = EPW // CHA

_f32 = jnp.float32


def _rsqrt16(x):
    i = lax.bitcast_convert_type(x, jnp.int32)
    i = jnp.int32(0x5F3759DF) - lax.shift_right_logical(i, 1)
    y = lax.bitcast_convert_type(i, _f32)
    for _ in range(3):
        y = y * (1.5 - 0.5 * x * y * y)
    return y


def _prep_body(row_hbm, deg_hbm, dis_hbm, disrow_hbm,
               idx0, idx1, idx2, idx3, idx4, ones_v, nslice_v, dslice_v,
               dis_v, rowe_v, dre_v, shared,
               si0, si1, si2, si3, si4, so0, so1, so2, so3, so4, sem):
    c = lax.axis_index("c")
    s = lax.axis_index("s")
    wid = c * NS + s
    idx = [idx0, idx1, idx2, idx3, idx4]
    si = [si0, si1, si2, si3, si4]
    so = [so0, so1, so2, so3, so4]
    nch = EPT // CH

    def zb(k, _):
        dslice_v[pl.ds(k * 16, 16)] = jnp.zeros((16,), _f32)
        return 0
    lax.fori_loop(0, NPT // 16, zb, 0)
    pltpu.sync_copy(dslice_v, shared.at[pl.ds(s * NPT, NPT)])
    for k in range(CH // 16):
        ones_v[pl.ds(k * 16, 16)] = jnp.ones((16,), _f32)
    plsc.subcore_barrier()

    def _start_load(k, j):
        pltpu.async_copy(row_hbm.at[pl.ds(s * EPT + k * CH, CH)],
                         idx[j], si[j])

    for j in range(4):
        _start_load(j, j)

    def deg_chunk(m, _):
        for j in range(5):
            k = 5 * m + j
            j1 = (j + 4) % 5
            pltpu.make_async_copy(row_hbm.at[pl.ds(0, CH)], idx[j],
                                  si[j]).wait()
            pltpu.async_copy(ones_v, shared.at[idx[j]], so[j], add=True)

            @pl.when(k >= 1)
            def _():
                pltpu.make_async_copy(ones_v, shared.at[pl.ds(0, CH)],
                                      so[j1]).wait()

            @pl.when(k + 4 < nch)
            def _():
                _start_load(k + 4, j1)
        return 0
    lax.fori_loop(0, nch // 5, deg_chunk, 0)
    pltpu.make_async_copy(ones_v, shared.at[pl.ds(0, CH)], so[4]).wait()
    plsc.subcore_barrier()

    pltpu.sync_copy(shared.at[pl.ds(s * NPT, NPT)], nslice_v)

    def rs(k, _):
        dg = nslice_v[pl.ds(k * 16, 16)] + 1.0
        nslice_v[pl.ds(k * 16, 16)] = dg
        dslice_v[pl.ds(k * 16, 16)] = _rsqrt16(dg)
        return 0
    lax.fori_loop(0, NPT // 16, rs, 0)

    @pl.when(c == 0)
    def _():
        pltpu.sync_copy(nslice_v, deg_hbm.at[pl.ds(s * NPT, NPT)])
        pltpu.sync_copy(dslice_v, dis_hbm.at[pl.ds(s * NPT, NPT)])

    pltpu.sync_copy(dslice_v, shared.at[pl.ds(s * NPT, NPT)])
    plsc.subcore_barrier()
    pltpu.sync_copy(shared, dis_v)
    pltpu.sync_copy(row_hbm.at[pl.ds(wid * EPW, EPW)], rowe_v)

    def gth(k, _):
        idx16 = rowe_v[pl.ds(k * 16, 16)]
        dre_v[pl.ds(k * 16, 16)] = plsc.load_gather(dis_v, [idx16])
        return 0
    lax.fori_loop(0, EPW // 16, gth, 0)
    pltpu.sync_copy(dre_v, disrow_hbm.at[pl.ds(wid * EPW, EPW)])


def _prep_call(row):
    return pl.kernel(
        _prep_body,
        out_type=[jax.ShapeDtypeStruct((NP,), _f32),
                  jax.ShapeDtypeStruct((NP,), _f32),
                  jax.ShapeDtypeStruct((E,), _f32)],
        mesh=plsc.VectorSubcoreMesh(core_axis_name="c", subcore_axis_name="s"),
        compiler_params=pltpu.CompilerParams(needs_layout_passes=False),
        scratch_types=(
            [pltpu.VMEM((CH,), jnp.int32)] * 5
            + [pltpu.VMEM((CH,), _f32),
               pltpu.VMEM((NPT,), _f32),
               pltpu.VMEM((NPT,), _f32),
               pltpu.VMEM((NP,), _f32),
               pltpu.VMEM((EPW,), jnp.int32),
               pltpu.VMEM((EPW,), _f32),
               pltpu.VMEM_SHARED((NP,), _f32)]
            + [pltpu.SemaphoreType.DMA] * 11
        ),
    )(row)


def _agg_body(layer, with_gather, hl2_hbm, ee_hbm, row_hbm, col_hbm,
              agg0_hbm, agg1_hbm,
              rowall, c0, c1, g0, g1, e0, e1, o0, o1, shared,
              sg0, sg1, se0, se1, sc0, sc1, sc2, sc3, so0, so1):
    c = lax.axis_index("c")
    s = lax.axis_index("s")
    wid = c * NS + s
    colv = [c0, c1]
    g = [g0, g1]
    ev = [e0, e1]
    ov = [o0, o1]
    sg = [sg0, sg1]
    se = [se0, se1]
    sc = [sc0, sc1, sc2, sc3]
    so = [so0, so1]
    ebase = wid * EPW

    def zg(i, _):
        for j in range(D // 16):
            g0[i, pl.ds(j * 16, 16)] = jnp.zeros((16,), _f32)
        return 0
    lax.fori_loop(0, CHA, zg, 0)
    nz = NPT // CHA
    for m in range(nz):
        j = m % 4
        if m >= 4:
            pltpu.make_async_copy(g0, shared.at[pl.ds(0, CHA)], sc[j]).wait()
        pltpu.async_copy(g0, shared.at[pl.ds(s * NPT + m * CHA, CHA)], sc[j])
    for j in range(4):
        pltpu.make_async_copy(g0, shared.at[pl.ds(0, CHA)], sc[j]).wait()

    pltpu.async_copy(row_hbm.at[pl.ds(ebase, EPW)], rowall, sg0).wait()
    plsc.subcore_barrier()

    def _start_in(k, b):
        if with_gather:
            pltpu.async_copy(
                hl2_hbm.at[rowall.at[pl.ds(k * CHA, CHA)]], g[b], sg[b])
        pltpu.async_copy(
            ee_hbm.at[layer, pl.ds(ebase + k * CHA, CHA)], ev[b], se[b])

    _start_in(0, 0)
    _start_in(1, 1)

    def chunk(k2, _):
        for b in range(2):
            k = 2 * k2 + b
            if with_gather:
                pltpu.make_async_copy(hl2_hbm.at[pl.ds(0, CHA)], g[b],
                                      sg[b]).wait()
            pltpu.make_async_copy(hl2_hbm.at[pl.ds(0, CHA)], ev[b],
                                  se[b]).wait()

            @pl.when(k2 >= 1)
            def _():
                pltpu.make_async_copy(ov[b], shared.at[pl.ds(0, CHA)],
                                      so[b]).wait()
            pltpu.async_copy(col_hbm.at[pl.ds(ebase + k * CHA, CHA)],
                             colv[b], sc[b])

            def edge(i, _):
                for jj in range(D // 16):
                    bb = ev[b][i, pl.ds(jj * 16, 16)]
                    if with_gather:
                        bb = g[b][i, pl.ds(jj * 16, 16)] + bb
                    ov[b][i, pl.ds(jj * 16, 16)] = jnp.maximum(bb, 0.0)
                return 0
            lax.fori_loop(0, CHA, edge, 0)

            @pl.when(k2 < NCHA // 2 - 1)
            def _():
                _start_in(k + 2, b)
            pltpu.make_async_copy(col_hbm.at[pl.ds(0, CHA)], colv[b],
                                  sc[b]).wait()
            pltpu.async_copy(ov[b], shared.at[colv[b]], so[b], add=True)
        return 0
    lax.fori_loop(0, NCHA // 2, chunk, 0)
    for b in range(2):
        pltpu.make_async_copy(ov[b], shared.at[pl.ds(0, CHA)], so[b]).wait()
    plsc.subcore_barrier()

    bufs = [g0, g1, o0, o1]
    sa = [sg0, sg1, se0, se1]
    nd = NPT // CHA
    for m in range(nd):
        j = m % 4
        if m >= 4:
            pltpu.make_async_copy(bufs[j], agg0_hbm.at[pl.ds(0, CHA)],
                                  sc[j]).wait()
        pltpu.async_copy(shared.at[pl.ds(s * NPT + m * CHA, CHA)],
                         bufs[j], sa[j]).wait()

        @pl.when(c == 0)
        def _():
            pltpu.async_copy(bufs[j],
                             agg0_hbm.at[pl.ds(s * NPT + m * CHA, CHA)],
                             sc[j])

        @pl.when(c == 1)
        def _():
            pltpu.async_copy(bufs[j],
                             agg1_hbm.at[pl.ds(s * NPT + m * CHA, CHA)],
                             sc[j])
    for j in range(4):
        pltpu.make_async_copy(bufs[j], agg0_hbm.at[pl.ds(0, CHA)],
                              sc[j]).wait()


def _agg_call(layer, with_gather, hl2, ee2, row, col):
    return pl.kernel(
        functools.partial(_agg_body, layer, with_gather),
        out_type=[jax.ShapeDtypeStruct((NP, D), _f32),
                  jax.ShapeDtypeStruct((NP, D), _f32)],
        mesh=plsc.VectorSubcoreMesh(core_axis_name="c", subcore_axis_name="s"),
        compiler_params=pltpu.CompilerParams(needs_layout_passes=False),
        scratch_types=(
            [pltpu.VMEM((EPW,), jnp.int32),
             pltpu.VMEM((CHA,), jnp.int32),
             pltpu.VMEM((CHA,), jnp.int32)]
            + [pltpu.VMEM((CHA, D), _f32)] * 6
            + [pltpu.VMEM_SHARED((NP, D), _f32)]
            + [pltpu.SemaphoreType.DMA] * 10
        ),
    )(hl2, ee2, row, col)


def _ee_body(be, ea_ref, w_ref, o_ref):
    o_ref[0] = jnp.dot(ea_ref[...], w_ref[0], preferred_element_type=_f32)


def _ee_call(ea_pad, w_all):
    be = 4000
    return pl.pallas_call(
        functools.partial(_ee_body, be),
        grid=(L, E // be),
        in_specs=[
            pl.BlockSpec((be, 8), lambda l, e: (e, 0)),
            pl.BlockSpec((1, 8, D), lambda l, e: (l, 0, 0)),
        ],
        out_specs=pl.BlockSpec((1, be, D), lambda l, e: (l, e, 0)),
        out_shape=jax.ShapeDtypeStruct((L, E, D), _f32),
    )(ea_pad, w_all)


def _node0_body(emb_ref, dis_ref, w_ref, b_ref, h_ref, hl_ref, hl2_ref):
    h = jnp.broadcast_to(emb_ref[...], (N, D))
    hl = jnp.dot(h, w_ref[...], preferred_element_type=_f32) + b_ref[...]
    h_ref[...] = h
    hl_ref[...] = hl
    hl2_ref[0:N] = dis_ref[...] * hl
    hl2_ref[N:NP] = jnp.zeros((NP - N, D), _f32)


def _node0_call(emb, dis, w0t, b0):
    return pl.pallas_call(
        _node0_body,
        out_shape=[jax.ShapeDtypeStruct((N, D), _f32),
                   jax.ShapeDtypeStruct((N, D), _f32),
                   jax.ShapeDtypeStruct((NP, D), _f32)],
    )(emb, dis, w0t, b0)


def _bn_update(agg0, agg1, hl_prev, h_prev, deg, dis, root, gamma, beta,
               do_relu):
    agg = dis * (agg0[0:N] + agg1[0:N])
    out = agg + jnp.maximum(hl_prev + root, 0.0) / deg
    mu = jnp.mean(out, axis=0, keepdims=True)
    var = jnp.mean((out - mu) ** 2, axis=0, keepdims=True)
    out = (out - mu) * jax.lax.rsqrt(var + 1e-5) * gamma + beta
    if do_relu:
        out = jnp.maximum(out, 0.0)
    return out + h_prev


def _node_body(do_relu, agg0_ref, agg1_ref, hlp_ref, hp_ref, deg_ref, dis_ref,
               root_ref, gamma_ref, beta_ref, w_ref, b_ref,
               h_ref, hl_ref, hl2_ref):
    h = _bn_update(agg0_ref[...], agg1_ref[...], hlp_ref[...], hp_ref[...],
                   deg_ref[...], dis_ref[...], root_ref[...], gamma_ref[...],
                   beta_ref[...], do_relu)
    hl = jnp.dot(h, w_ref[...], preferred_element_type=_f32) + b_ref[...]
    h_ref[...] = h
    hl_ref[...] = hl
    hl2_ref[0:N] = dis_ref[...] * hl
    hl2_ref[N:NP] = jnp.zeros((NP - N, D), _f32)


def _node_call(do_relu, agg0, agg1, hl_prev, h_prev, deg, dis, root, gamma,
               beta, wt, b):
    return pl.pallas_call(
        functools.partial(_node_body, do_relu),
        out_shape=[jax.ShapeDtypeStruct((N, D), _f32),
                   jax.ShapeDtypeStruct((N, D), _f32),
                   jax.ShapeDtypeStruct((NP, D), _f32)],
    )(agg0, agg1, hl_prev, h_prev, deg, dis, root, gamma, beta, wt, b)


def _final_body(agg0_ref, agg1_ref, hlp_ref, hp_ref, deg_ref, dis_ref,
                root_ref, gamma_ref, beta_ref, batch_ref, wp_ref, bp_ref,
                o_ref):
    h = _bn_update(agg0_ref[...], agg1_ref[...], hlp_ref[...], hp_ref[...],
                   deg_ref[...], dis_ref[...], root_ref[...], gamma_ref[...],
                   beta_ref[...], do_relu=False)
    gids = lax.broadcasted_iota(jnp.int32, (1, G), 1)
    m = (batch_ref[...] == gids).astype(_f32)
    hg_sum = lax.dot_general(m, h, (((0,), (0,)), ((), ())),
                             preferred_element_type=_f32)
    counts = lax.dot_general(m, jnp.ones((N, 1), _f32),
                             (((0,), (0,)), ((), ())),
                             preferred_element_type=_f32)
    hg = hg_sum / jnp.maximum(counts, 1.0)
    o_ref[...] = jnp.dot(hg, wp_ref[...], preferred_element_type=_f32) \
        + bp_ref[...]


def _final_call(agg0, agg1, hl_prev, h_prev, deg, dis, root, gamma, beta,
                batch, wpt, bp):
    out_dim = wpt.shape[1]
    return pl.pallas_call(
        _final_body,
        out_shape=jax.ShapeDtypeStruct((G, out_dim), _f32),
    )(agg0, agg1, hl_prev, h_prev, deg, dis, root, gamma, beta, batch, wpt, bp)


def kernel(x, edge_index, edge_attr, batch, params):
    row = edge_index[0]
    col = edge_index[1]

    deg_p, dis_p, dis_row = _prep_call(row)
    deg = deg_p[:N].reshape(N, 1)
    dis = dis_p[:N].reshape(N, 1)

    hl0row = params['node_emb'][0] @ params['W_lin'][0].T + params['b_lin'][0]
    ea_pad = jnp.concatenate([edge_attr, jnp.ones((E, 1), _f32)], axis=1)
    ea2c = ea_pad * dis_row[:, None]
    bias = params['b_edge'].at[0].add(hl0row)
    w_all = jnp.concatenate(
        [jnp.transpose(params['W_edge'], (0, 2, 1)),
         bias[:, None, :]], axis=1)
    ee2 = _ee_call(ea2c, w_all)

    h, hl, hl2 = _node0_call(params['node_emb'], dis,
                             params['W_lin'][0].T, params['b_lin'][0][None])

    for l in range(1, L):
        agg0, agg1 = _agg_call(l - 1, l - 1 > 0, hl2, ee2, row, col)
        h, hl, hl2 = _node_call(
            l - 1 < L - 1, agg0, agg1, hl, h, deg, dis,
            params['root'][l - 1], params['gamma'][l - 1][None],
            params['beta'][l - 1][None],
            params['W_lin'][l].T, params['b_lin'][l][None])

    agg0, agg1 = _agg_call(L - 1, True, hl2, ee2, row, col)
    return _final_call(agg0, agg1, hl, h, deg, dis,
                       params['root'][L - 1], params['gamma'][L - 1][None],
                       params['beta'][L - 1][None],
                       batch.reshape(N, 1), params['W_pred'].T,
                       params['b_pred'][None])


# speedup vs baseline: 1.5887x; 1.1630x over previous
import functools

import jax
import jax.numpy as jnp
from jax import lax
from jax.experimental import pallas as pl
from jax.experimental.pallas import tpu as pltpu
from jax.experimental.pallas import tpu_sc as plsc

N = 10000
NP = 10240
E = 320000
D = 128
G = 64
L = 4

NC = 2
NS = 16
NW = NC * NS
NPT = NP // NS
EPW = E // NW
EPT = E // NS
CH = 80
CHA = 40
NCHA = EPW // CHA

_f32 = jnp.float32


def _rsqrt16(x):
    i = lax.bitcast_convert_type(x, jnp.int32)
    i = jnp.int32(0x5F3759DF) - lax.shift_right_logical(i, 1)
    y = lax.bitcast_convert_type(i, _f32)
    for _ in range(3):
        y = y * (1.5 - 0.5 * x * y * y)
    return y


def _prep_body(row_hbm, deg_hbm, dis_hbm, disrow_hbm,
               idx0, idx1, idx2, idx3, idx4, ones_v, nslice_v, dslice_v,
               dis_v, rowe_v, dre_v, shared,
               si0, si1, si2, si3, si4, so0, so1, so2, so3, so4, sem):
    c = lax.axis_index("c")
    s = lax.axis_index("s")
    wid = c * NS + s
    idx = [idx0, idx1, idx2, idx3, idx4]
    si = [si0, si1, si2, si3, si4]
    so = [so0, so1, so2, so3, so4]
    nch = EPT // CH

    def zb(k, _):
        dslice_v[pl.ds(k * 16, 16)] = jnp.zeros((16,), _f32)
        return 0
    lax.fori_loop(0, NPT // 16, zb, 0)
    pltpu.sync_copy(dslice_v, shared.at[pl.ds(s * NPT, NPT)])
    for k in range(CH // 16):
        ones_v[pl.ds(k * 16, 16)] = jnp.ones((16,), _f32)
    plsc.subcore_barrier()

    def _start_load(k, j):
        pltpu.async_copy(row_hbm.at[pl.ds(s * EPT + k * CH, CH)],
                         idx[j], si[j])

    for j in range(4):
        _start_load(j, j)

    def deg_chunk(m, _):
        for j in range(5):
            k = 5 * m + j
            j1 = (j + 4) % 5
            pltpu.make_async_copy(row_hbm.at[pl.ds(0, CH)], idx[j],
                                  si[j]).wait()
            pltpu.async_copy(ones_v, shared.at[idx[j]], so[j], add=True)

            @pl.when(k >= 1)
            def _():
                pltpu.make_async_copy(ones_v, shared.at[pl.ds(0, CH)],
                                      so[j1]).wait()

            @pl.when(k + 4 < nch)
            def _():
                _start_load(k + 4, j1)
        return 0
    lax.fori_loop(0, nch // 5, deg_chunk, 0)
    pltpu.make_async_copy(ones_v, shared.at[pl.ds(0, CH)], so[4]).wait()
    plsc.subcore_barrier()

    pltpu.sync_copy(shared.at[pl.ds(s * NPT, NPT)], nslice_v)

    def rs(k, _):
        dg = nslice_v[pl.ds(k * 16, 16)] + 1.0
        nslice_v[pl.ds(k * 16, 16)] = dg
        dslice_v[pl.ds(k * 16, 16)] = _rsqrt16(dg)
        return 0
    lax.fori_loop(0, NPT // 16, rs, 0)

    @pl.when(c == 0)
    def _():
        pltpu.sync_copy(nslice_v, deg_hbm.at[pl.ds(s * NPT, NPT)])
        pltpu.sync_copy(dslice_v, dis_hbm.at[pl.ds(s * NPT, NPT)])

    pltpu.sync_copy(dslice_v, shared.at[pl.ds(s * NPT, NPT)])
    plsc.subcore_barrier()
    pltpu.sync_copy(shared, dis_v)
    pltpu.sync_copy(row_hbm.at[pl.ds(wid * EPW, EPW)], rowe_v)

    def gth(k, _):
        idx16 = rowe_v[pl.ds(k * 16, 16)]
        dre_v[pl.ds(k * 16, 16)] = plsc.load_gather(dis_v, [idx16])
        return 0
    lax.fori_loop(0, EPW // 16, gth, 0)
    pltpu.sync_copy(dre_v, disrow_hbm.at[pl.ds(wid * EPW, EPW)])


def _prep_call(row):
    return pl.kernel(
        _prep_body,
        out_type=[jax.ShapeDtypeStruct((NP,), _f32),
                  jax.ShapeDtypeStruct((NP,), _f32),
                  jax.ShapeDtypeStruct((E,), _f32)],
        mesh=plsc.VectorSubcoreMesh(core_axis_name="c", subcore_axis_name="s"),
        compiler_params=pltpu.CompilerParams(needs_layout_passes=False),
        scratch_types=(
            [pltpu.VMEM((CH,), jnp.int32)] * 5
            + [pltpu.VMEM((CH,), _f32),
               pltpu.VMEM((NPT,), _f32),
               pltpu.VMEM((NPT,), _f32),
               pltpu.VMEM((NP,), _f32),
               pltpu.VMEM((EPW,), jnp.int32),
               pltpu.VMEM((EPW,), _f32),
               pltpu.VMEM_SHARED((NP,), _f32)]
            + [pltpu.SemaphoreType.DMA] * 11
        ),
    )(row)


def _agg_body(layer, with_gather, hl2_hbm, ee_hbm, row_hbm, col_hbm,
              agg0_hbm, agg1_hbm,
              rowall, c0, c1, g0, g1, e0, e1, o0, o1, shared,
              sg0, sg1, se0, se1, sc0, sc1, sc2, sc3, so0, so1):
    c = lax.axis_index("c")
    s = lax.axis_index("s")
    wid = c * NS + s
    colv = [c0, c1]
    g = [g0, g1]
    ev = [e0, e1]
    ov = [o0, o1]
    sg = [sg0, sg1]
    se = [se0, se1]
    sc = [sc0, sc1, sc2, sc3]
    so = [so0, so1]
    ebase = wid * EPW

    def zg(i, _):
        for j in range(D // 16):
            g0[i, pl.ds(j * 16, 16)] = jnp.zeros((16,), _f32)
        return 0
    lax.fori_loop(0, CHA, zg, 0)
    nz = NPT // CHA
    for m in range(nz):
        j = m % 4
        if m >= 4:
            pltpu.make_async_copy(g0, shared.at[pl.ds(0, CHA)], sc[j]).wait()
        pltpu.async_copy(g0, shared.at[pl.ds(s * NPT + m * CHA, CHA)], sc[j])
    for j in range(4):
        pltpu.make_async_copy(g0, shared.at[pl.ds(0, CHA)], sc[j]).wait()

    pltpu.async_copy(row_hbm.at[pl.ds(ebase, EPW)], rowall, sg0).wait()
    plsc.subcore_barrier()

    def _start_in(k, b):
        if with_gather:
            pltpu.async_copy(
                hl2_hbm.at[rowall.at[pl.ds(k * CHA, CHA)]], g[b], sg[b])
        pltpu.async_copy(
            ee_hbm.at[layer, pl.ds(ebase + k * CHA, CHA)], ev[b], se[b])

    _start_in(0, 0)
    _start_in(1, 1)

    def chunk(k2, _):
        for b in range(2):
            k = 2 * k2 + b
            if with_gather:
                pltpu.make_async_copy(hl2_hbm.at[pl.ds(0, CHA)], g[b],
                                      sg[b]).wait()
            pltpu.make_async_copy(hl2_hbm.at[pl.ds(0, CHA)], ev[b],
                                  se[b]).wait()

            @pl.when(k2 >= 1)
            def _():
                pltpu.make_async_copy(ov[b], shared.at[pl.ds(0, CHA)],
                                      so[b]).wait()
            pltpu.async_copy(col_hbm.at[pl.ds(ebase + k * CHA, CHA)],
                             colv[b], sc[b])

            def edge(i, _):
                for jj in range(D // 16):
                    bb = ev[b][i, pl.ds(jj * 16, 16)]
                    if with_gather:
                        bb = g[b][i, pl.ds(jj * 16, 16)] + bb
                    ov[b][i, pl.ds(jj * 16, 16)] = jnp.maximum(bb, 0.0)
                return 0
            lax.fori_loop(0, CHA, edge, 0)

            @pl.when(k2 < NCHA // 2 - 1)
            def _():
                _start_in(k + 2, b)
            pltpu.make_async_copy(col_hbm.at[pl.ds(0, CHA)], colv[b],
                                  sc[b]).wait()
            pltpu.async_copy(ov[b], shared.at[colv[b]], so[b], add=True)
        return 0
    lax.fori_loop(0, NCHA // 2, chunk, 0)
    for b in range(2):
        pltpu.make_async_copy(ov[b], shared.at[pl.ds(0, CHA)], so[b]).wait()
    plsc.subcore_barrier()

    bufs = [g0, g1, o0, o1]
    sa = [sg0, sg1, se0, se1]
    nd = NPT // CHA
    for m in range(nd):
        j = m % 4
        if m >= 4:
            pltpu.make_async_copy(bufs[j], agg0_hbm.at[pl.ds(0, CHA)],
                                  sc[j]).wait()
        pltpu.async_copy(shared.at[pl.ds(s * NPT + m * CHA, CHA)],
                         bufs[j], sa[j]).wait()

        @pl.when(c == 0)
        def _():
            pltpu.async_copy(bufs[j],
                             agg0_hbm.at[pl.ds(s * NPT + m * CHA, CHA)],
                             sc[j])

        @pl.when(c == 1)
        def _():
            pltpu.async_copy(bufs[j],
                             agg1_hbm.at[pl.ds(s * NPT + m * CHA, CHA)],
                             sc[j])
    for j in range(4):
        pltpu.make_async_copy(bufs[j], agg0_hbm.at[pl.ds(0, CHA)],
                              sc[j]).wait()


def _agg_call(layer, with_gather, hl2, ee2, row, col):
    return pl.kernel(
        functools.partial(_agg_body, layer, with_gather),
        out_type=[jax.ShapeDtypeStruct((NP, D), _f32),
                  jax.ShapeDtypeStruct((NP, D), _f32)],
        mesh=plsc.VectorSubcoreMesh(core_axis_name="c", subcore_axis_name="s"),
        compiler_params=pltpu.CompilerParams(needs_layout_passes=False),
        scratch_types=(
            [pltpu.VMEM((EPW,), jnp.int32),
             pltpu.VMEM((CHA,), jnp.int32),
             pltpu.VMEM((CHA,), jnp.int32)]
            + [pltpu.VMEM((CHA, D), _f32)] * 6
            + [pltpu.VMEM_SHARED((NP, D), _f32)]
            + [pltpu.SemaphoreType.DMA] * 10
        ),
    )(hl2, ee2, row, col)


def _ee_body(ea_ref, w_ref, o_ref):
    ea = ea_ref[...]
    for l in range(L):
        o_ref[l] = jnp.dot(ea, w_ref[l], preferred_element_type=_f32)


def _ee_call(ea_pad, w_all):
    be = 4000
    return pl.pallas_call(
        _ee_body,
        grid=(E // be,),
        in_specs=[
            pl.BlockSpec((be, 8), lambda e: (e, 0)),
            pl.BlockSpec((L, 8, D), lambda e: (0, 0, 0)),
        ],
        out_specs=pl.BlockSpec((L, be, D), lambda e: (0, e, 0)),
        out_shape=jax.ShapeDtypeStruct((L, E, D), _f32),
    )(ea_pad, w_all)


def _node0_body(emb_ref, dis_ref, w_ref, b_ref, h_ref, hl_ref, hl2_ref):
    h = jnp.broadcast_to(emb_ref[...], (N, D))
    hl = jnp.dot(h, w_ref[...], preferred_element_type=_f32) + b_ref[...]
    h_ref[...] = h
    hl_ref[...] = hl
    hl2_ref[0:N] = dis_ref[...] * hl
    hl2_ref[N:NP] = jnp.zeros((NP - N, D), _f32)


def _node0_call(emb, dis, w0t, b0):
    return pl.pallas_call(
        _node0_body,
        out_shape=[jax.ShapeDtypeStruct((N, D), _f32),
                   jax.ShapeDtypeStruct((N, D), _f32),
                   jax.ShapeDtypeStruct((NP, D), _f32)],
    )(emb, dis, w0t, b0)


def _bn_update(agg0, agg1, hl_prev, h_prev, deg, dis, root, gamma, beta,
               do_relu):
    agg = dis * (agg0[0:N] + agg1[0:N])
    out = agg + jnp.maximum(hl_prev + root, 0.0) / deg
    mu = jnp.mean(out, axis=0, keepdims=True)
    var = jnp.mean((out - mu) ** 2, axis=0, keepdims=True)
    out = (out - mu) * jax.lax.rsqrt(var + 1e-5) * gamma + beta
    if do_relu:
        out = jnp.maximum(out, 0.0)
    return out + h_prev


def _node_body(do_relu, agg0_ref, agg1_ref, hlp_ref, hp_ref, deg_ref, dis_ref,
               root_ref, gamma_ref, beta_ref, w_ref, b_ref,
               h_ref, hl_ref, hl2_ref):
    h = _bn_update(agg0_ref[...], agg1_ref[...], hlp_ref[...], hp_ref[...],
                   deg_ref[...], dis_ref[...], root_ref[...], gamma_ref[...],
                   beta_ref[...], do_relu)
    hl = jnp.dot(h, w_ref[...], preferred_element_type=_f32) + b_ref[...]
    h_ref[...] = h
    hl_ref[...] = hl
    hl2_ref[0:N] = dis_ref[...] * hl
    hl2_ref[N:NP] = jnp.zeros((NP - N, D), _f32)


def _node_call(do_relu, agg0, agg1, hl_prev, h_prev, deg, dis, root, gamma,
               beta, wt, b):
    return pl.pallas_call(
        functools.partial(_node_body, do_relu),
        out_shape=[jax.ShapeDtypeStruct((N, D), _f32),
                   jax.ShapeDtypeStruct((N, D), _f32),
                   jax.ShapeDtypeStruct((NP, D), _f32)],
    )(agg0, agg1, hl_prev, h_prev, deg, dis, root, gamma, beta, wt, b)


def _final_body(agg0_ref, agg1_ref, hlp_ref, hp_ref, deg_ref, dis_ref,
                root_ref, gamma_ref, beta_ref, batch_ref, wp_ref, bp_ref,
                o_ref):
    h = _bn_update(agg0_ref[...], agg1_ref[...], hlp_ref[...], hp_ref[...],
                   deg_ref[...], dis_ref[...], root_ref[...], gamma_ref[...],
                   beta_ref[...], do_relu=False)
    gids = lax.broadcasted_iota(jnp.int32, (1, G), 1)
    m = (batch_ref[...] == gids).astype(_f32)
    hg_sum = lax.dot_general(m, h, (((0,), (0,)), ((), ())),
                             preferred_element_type=_f32)
    counts = lax.dot_general(m, jnp.ones((N, 1), _f32),
                             (((0,), (0,)), ((), ())),
                             preferred_element_type=_f32)
    hg = hg_sum / jnp.maximum(counts, 1.0)
    o_ref[...] = jnp.dot(hg, wp_ref[...], preferred_element_type=_f32) \
        + bp_ref[...]


def _final_call(agg0, agg1, hl_prev, h_prev, deg, dis, root, gamma, beta,
                batch, wpt, bp):
    out_dim = wpt.shape[1]
    return pl.pallas_call(
        _final_body,
        out_shape=jax.ShapeDtypeStruct((G, out_dim), _f32),
    )(agg0, agg1, hl_prev, h_prev, deg, dis, root, gamma, beta, batch, wpt, bp)


def kernel(x, edge_index, edge_attr, batch, params):
    row = edge_index[0]
    col = edge_index[1]

    deg_p, dis_p, dis_row = _prep_call(row)
    deg = deg_p[:N].reshape(N, 1)
    dis = dis_p[:N].reshape(N, 1)

    hl0row = params['node_emb'][0] @ params['W_lin'][0].T + params['b_lin'][0]
    ea_pad = jnp.concatenate([edge_attr, jnp.ones((E, 1), _f32)], axis=1)
    ea2c = ea_pad * dis_row[:, None]
    bias = params['b_edge'].at[0].add(hl0row)
    w_all = jnp.concatenate(
        [jnp.transpose(params['W_edge'], (0, 2, 1)),
         bias[:, None, :]], axis=1)
    ee2 = _ee_call(ea2c, w_all)

    h, hl, hl2 = _node0_call(params['node_emb'], dis,
                             params['W_lin'][0].T, params['b_lin'][0][None])

    for l in range(1, L):
        agg0, agg1 = _agg_call(l - 1, l - 1 > 0, hl2, ee2, row, col)
        h, hl, hl2 = _node_call(
            l - 1 < L - 1, agg0, agg1, hl, h, deg, dis,
            params['root'][l - 1], params['gamma'][l - 1][None],
            params['beta'][l - 1][None],
            params['W_lin'][l].T, params['b_lin'][l][None])

    agg0, agg1 = _agg_call(L - 1, True, hl2, ee2, row, col)
    return _final_call(agg0, agg1, hl, h, deg, dis,
                       params['root'][L - 1], params['gamma'][L - 1][None],
                       params['beta'][L - 1][None],
                       batch.reshape(N, 1), params['W_pred'].T,
                       params['b_pred'][None])


# speedup vs baseline: 1.7133x; 1.0784x over previous
import functools

import jax
import jax.numpy as jnp
from jax import lax
from jax.experimental import pallas as pl
from jax.experimental.pallas import tpu as pltpu
from jax.experimental.pallas import tpu_sc as plsc

N = 10000
NP = 10240
E = 320000
D = 128
G = 64
L = 4

NC = 2
NS = 16
NW = NC * NS
NPT = NP // NS
EPW = E // NW
EPT = E // NS
CH = 80
CHA = 40
NCHA = EPW // CHA

_f32 = jnp.float32


def _rsqrt16(x):
    i = lax.bitcast_convert_type(x, jnp.int32)
    i = jnp.int32(0x5F3759DF) - lax.shift_right_logical(i, 1)
    y = lax.bitcast_convert_type(i, _f32)
    for _ in range(3):
        y = y * (1.5 - 0.5 * x * y * y)
    return y


def _prep_body(row_hbm, deg_hbm, dis_hbm, disrow_hbm,
               idx0, idx1, idx2, idx3, idx4, ones_v, nslice_v, dslice_v,
               dis_v, rowe_v, dre_v, shared,
               si0, si1, si2, si3, si4, so0, so1, so2, so3, so4, sem):
    c = lax.axis_index("c")
    s = lax.axis_index("s")
    wid = c * NS + s
    idx = [idx0, idx1, idx2, idx3, idx4]
    si = [si0, si1, si2, si3, si4]
    so = [so0, so1, so2, so3, so4]
    nch = EPT // CH

    def zb(k, _):
        dslice_v[pl.ds(k * 16, 16)] = jnp.zeros((16,), _f32)
        return 0
    lax.fori_loop(0, NPT // 16, zb, 0)
    pltpu.sync_copy(dslice_v, shared.at[pl.ds(s * NPT, NPT)])
    for k in range(CH // 16):
        ones_v[pl.ds(k * 16, 16)] = jnp.ones((16,), _f32)
    plsc.subcore_barrier()

    def _start_load(k, j):
        pltpu.async_copy(row_hbm.at[pl.ds(s * EPT + k * CH, CH)],
                         idx[j], si[j])

    for j in range(4):
        _start_load(j, j)

    def deg_chunk(m, _):
        for j in range(5):
            k = 5 * m + j
            j1 = (j + 4) % 5
            pltpu.make_async_copy(row_hbm.at[pl.ds(0, CH)], idx[j],
                                  si[j]).wait()
            pltpu.async_copy(ones_v, shared.at[idx[j]], so[j], add=True)

            @pl.when(k >= 1)
            def _():
                pltpu.make_async_copy(ones_v, shared.at[pl.ds(0, CH)],
                                      so[j1]).wait()

            @pl.when(k + 4 < nch)
            def _():
                _start_load(k + 4, j1)
        return 0
    lax.fori_loop(0, nch // 5, deg_chunk, 0)
    pltpu.make_async_copy(ones_v, shared.at[pl.ds(0, CH)], so[4]).wait()
    plsc.subcore_barrier()

    pltpu.sync_copy(shared.at[pl.ds(s * NPT, NPT)], nslice_v)

    def rs(k, _):
        dg = nslice_v[pl.ds(k * 16, 16)] + 1.0
        nslice_v[pl.ds(k * 16, 16)] = dg
        dslice_v[pl.ds(k * 16, 16)] = _rsqrt16(dg)
        return 0
    lax.fori_loop(0, NPT // 16, rs, 0)

    @pl.when(c == 0)
    def _():
        pltpu.sync_copy(nslice_v, deg_hbm.at[pl.ds(s * NPT, NPT)])
        pltpu.sync_copy(dslice_v, dis_hbm.at[pl.ds(s * NPT, NPT)])

    pltpu.sync_copy(dslice_v, shared.at[pl.ds(s * NPT, NPT)])
    plsc.subcore_barrier()
    pltpu.sync_copy(shared, dis_v)
    pltpu.sync_copy(row_hbm.at[pl.ds(wid * EPW, EPW)], rowe_v)

    def gth(k, _):
        idx16 = rowe_v[pl.ds(k * 16, 16)]
        dre_v[pl.ds(k * 16, 16)] = plsc.load_gather(dis_v, [idx16])
        return 0
    lax.fori_loop(0, EPW // 16, gth, 0)
    pltpu.sync_copy(dre_v, disrow_hbm.at[pl.ds(wid * EPW, EPW)])


def _prep_call(row):
    return pl.kernel(
        _prep_body,
        out_type=[jax.ShapeDtypeStruct((NP,), _f32),
                  jax.ShapeDtypeStruct((NP,), _f32),
                  jax.ShapeDtypeStruct((E,), _f32)],
        mesh=plsc.VectorSubcoreMesh(core_axis_name="c", subcore_axis_name="s"),
        compiler_params=pltpu.CompilerParams(needs_layout_passes=False),
        scratch_types=(
            [pltpu.VMEM((CH,), jnp.int32)] * 5
            + [pltpu.VMEM((CH,), _f32),
               pltpu.VMEM((NPT,), _f32),
               pltpu.VMEM((NPT,), _f32),
               pltpu.VMEM((NP,), _f32),
               pltpu.VMEM((EPW,), jnp.int32),
               pltpu.VMEM((EPW,), _f32),
               pltpu.VMEM_SHARED((NP,), _f32)]
            + [pltpu.SemaphoreType.DMA] * 11
        ),
    )(row)


def _agg_body(layer, with_gather, hl2_hbm, ee_hbm, row_hbm, col_hbm,
              agg0_hbm, agg1_hbm,
              rowall, c0, c1, g0, g1, e0, e1, o0, o1, shared,
              sg0, sg1, se0, se1, sc0, sc1, sc2, sc3, so0, so1):
    c = lax.axis_index("c")
    s = lax.axis_index("s")
    wid = c * NS + s
    colv = [c0, c1]
    g = [g0, g1]
    ev = [e0, e1]
    ov = [o0, o1]
    sg = [sg0, sg1]
    se = [se0, se1]
    sc = [sc0, sc1, sc2, sc3]
    so = [so0, so1]
    ebase = wid * EPW

    def zg(i, _):
        for j in range(D // 16):
            g0[i, pl.ds(j * 16, 16)] = jnp.zeros((16,), _f32)
        return 0
    lax.fori_loop(0, CHA, zg, 0)
    nz = NPT // CHA
    for m in range(nz):
        j = m % 4
        if m >= 4:
            pltpu.make_async_copy(g0, shared.at[pl.ds(0, CHA)], sc[j]).wait()
        pltpu.async_copy(g0, shared.at[pl.ds(s * NPT + m * CHA, CHA)], sc[j])
    for j in range(4):
        pltpu.make_async_copy(g0, shared.at[pl.ds(0, CHA)], sc[j]).wait()

    pltpu.async_copy(row_hbm.at[pl.ds(ebase, EPW)], rowall, sg0).wait()
    plsc.subcore_barrier()

    def _start_in(k, b):
        if with_gather:
            pltpu.async_copy(
                hl2_hbm.at[rowall.at[pl.ds(k * CHA, CHA)]], g[b], sg[b])
        pltpu.async_copy(
            ee_hbm.at[layer, pl.ds(ebase + k * CHA, CHA)], ev[b], se[b])

    _start_in(0, 0)
    _start_in(1, 1)

    def chunk(k2, _):
        for b in range(2):
            k = 2 * k2 + b
            if with_gather:
                pltpu.make_async_copy(hl2_hbm.at[pl.ds(0, CHA)], g[b],
                                      sg[b]).wait()
            pltpu.make_async_copy(hl2_hbm.at[pl.ds(0, CHA)], ev[b],
                                  se[b]).wait()

            @pl.when(k2 >= 1)
            def _():
                pltpu.make_async_copy(ov[b], shared.at[pl.ds(0, CHA)],
                                      so[b]).wait()
            pltpu.async_copy(col_hbm.at[pl.ds(ebase + k * CHA, CHA)],
                             colv[b], sc[b])

            def edge(i, _):
                for jj in range(D // 16):
                    bb = ev[b][i, pl.ds(jj * 16, 16)]
                    if with_gather:
                        bb = g[b][i, pl.ds(jj * 16, 16)] + bb
                    ov[b][i, pl.ds(jj * 16, 16)] = jnp.maximum(bb, 0.0)
                return 0
            lax.fori_loop(0, CHA, edge, 0)

            @pl.when(k2 < NCHA // 2 - 1)
            def _():
                _start_in(k + 2, b)
            pltpu.make_async_copy(col_hbm.at[pl.ds(0, CHA)], colv[b],
                                  sc[b]).wait()
            pltpu.async_copy(ov[b], shared.at[colv[b]], so[b], add=True)
        return 0
    lax.fori_loop(0, NCHA // 2, chunk, 0)
    for b in range(2):
        pltpu.make_async_copy(ov[b], shared.at[pl.ds(0, CHA)], so[b]).wait()
    plsc.subcore_barrier()

    bufs = [g0, g1, o0, o1]
    sa = [sg0, sg1, se0, se1]
    nd = NPT // CHA
    for m in range(nd):
        j = m % 4
        if m >= 4:
            pltpu.make_async_copy(bufs[j], agg0_hbm.at[pl.ds(0, CHA)],
                                  sc[j]).wait()
        pltpu.async_copy(shared.at[pl.ds(s * NPT + m * CHA, CHA)],
                         bufs[j], sa[j]).wait()

        @pl.when(c == 0)
        def _():
            pltpu.async_copy(bufs[j],
                             agg0_hbm.at[pl.ds(s * NPT + m * CHA, CHA)],
                             sc[j])

        @pl.when(c == 1)
        def _():
            pltpu.async_copy(bufs[j],
                             agg1_hbm.at[pl.ds(s * NPT + m * CHA, CHA)],
                             sc[j])
    for j in range(4):
        pltpu.make_async_copy(bufs[j], agg0_hbm.at[pl.ds(0, CHA)],
                              sc[j]).wait()


def _agg_call(layer, with_gather, hl2, ee2, row, col):
    return pl.kernel(
        functools.partial(_agg_body, layer, with_gather),
        out_type=[jax.ShapeDtypeStruct((NP, D), _f32),
                  jax.ShapeDtypeStruct((NP, D), _f32)],
        mesh=plsc.VectorSubcoreMesh(core_axis_name="c", subcore_axis_name="s"),
        compiler_params=pltpu.CompilerParams(needs_layout_passes=False),
        scratch_types=(
            [pltpu.VMEM((EPW,), jnp.int32),
             pltpu.VMEM((CHA,), jnp.int32),
             pltpu.VMEM((CHA,), jnp.int32)]
            + [pltpu.VMEM((CHA, D), _f32)] * 6
            + [pltpu.VMEM_SHARED((NP, D), _f32)]
            + [pltpu.SemaphoreType.DMA] * 10
        ),
    )(hl2, ee2, row, col)


def _ee_body(be, ea_ref, dis_ref, w_ref, b_ref, o_ref):
    ea = ea_ref[...]
    ii = lax.broadcasted_iota(jnp.int32, (128, 128), 0)
    jj = lax.broadcasted_iota(jnp.int32, (128, 128), 1)
    eye = (ii == jj).astype(_f32)
    yt = lax.dot_general(eye, dis_ref[0], (((1,), (1,)), ((), ())),
                         preferred_element_type=_f32)
    for l in range(L):
        ee = jnp.dot(ea, w_ref[l], preferred_element_type=_f32) + b_ref[l]
        for s2 in range(be // 128):
            o_ref[l, 128 * s2:128 * (s2 + 1), :] = (
                ee[128 * s2:128 * (s2 + 1), :] * yt[:, s2:s2 + 1])


def _ee_call(edge_attr, dis_row2d, w7, b_all):
    be = 2560
    return pl.pallas_call(
        functools.partial(_ee_body, be),
        grid=(E // be,),
        in_specs=[
            pl.BlockSpec((be, 7), lambda e: (e, 0)),
            pl.BlockSpec((1, be // 128, 128), lambda e: (e, 0, 0)),
            pl.BlockSpec((L, 7, D), lambda e: (0, 0, 0)),
            pl.BlockSpec((L, 1, D), lambda e: (0, 0, 0)),
        ],
        out_specs=pl.BlockSpec((L, be, D), lambda e: (0, e, 0)),
        out_shape=jax.ShapeDtypeStruct((L, E, D), _f32),
    )(edge_attr, dis_row2d, w7, b_all)


def _node0_body(emb_ref, dis_ref, w_ref, b_ref, h_ref, hl_ref, hl2_ref):
    h = jnp.broadcast_to(emb_ref[...], (N, D))
    hl = jnp.dot(h, w_ref[...], preferred_element_type=_f32) + b_ref[...]
    h_ref[...] = h
    hl_ref[...] = hl
    hl2_ref[0:N] = dis_ref[...] * hl
    hl2_ref[N:NP] = jnp.zeros((NP - N, D), _f32)


def _node0_call(emb, dis, w0t, b0):
    return pl.pallas_call(
        _node0_body,
        out_shape=[jax.ShapeDtypeStruct((N, D), _f32),
                   jax.ShapeDtypeStruct((N, D), _f32),
                   jax.ShapeDtypeStruct((NP, D), _f32)],
    )(emb, dis, w0t, b0)


def _bn_update(agg0, agg1, hl_prev, h_prev, deg, dis, root, gamma, beta,
               do_relu):
    agg = dis * (agg0[0:N] + agg1[0:N])
    out = agg + jnp.maximum(hl_prev + root, 0.0) / deg
    mu = jnp.mean(out, axis=0, keepdims=True)
    var = jnp.mean((out - mu) ** 2, axis=0, keepdims=True)
    out = (out - mu) * jax.lax.rsqrt(var + 1e-5) * gamma + beta
    if do_relu:
        out = jnp.maximum(out, 0.0)
    return out + h_prev


def _node_body(do_relu, agg0_ref, agg1_ref, hlp_ref, hp_ref, deg_ref, dis_ref,
               root_ref, gamma_ref, beta_ref, w_ref, b_ref,
               h_ref, hl_ref, hl2_ref):
    h = _bn_update(agg0_ref[...], agg1_ref[...], hlp_ref[...], hp_ref[...],
                   deg_ref[...], dis_ref[...], root_ref[...], gamma_ref[...],
                   beta_ref[...], do_relu)
    hl = jnp.dot(h, w_ref[...], preferred_element_type=_f32) + b_ref[...]
    h_ref[...] = h
    hl_ref[...] = hl
    hl2_ref[0:N] = dis_ref[...] * hl
    hl2_ref[N:NP] = jnp.zeros((NP - N, D), _f32)


def _node_call(do_relu, agg0, agg1, hl_prev, h_prev, deg, dis, root, gamma,
               beta, wt, b):
    return pl.pallas_call(
        functools.partial(_node_body, do_relu),
        out_shape=[jax.ShapeDtypeStruct((N, D), _f32),
                   jax.ShapeDtypeStruct((N, D), _f32),
                   jax.ShapeDtypeStruct((NP, D), _f32)],
    )(agg0, agg1, hl_prev, h_prev, deg, dis, root, gamma, beta, wt, b)


def _final_body(agg0_ref, agg1_ref, hlp_ref, hp_ref, deg_ref, dis_ref,
                root_ref, gamma_ref, beta_ref, batch_ref, wp_ref, bp_ref,
                o_ref):
    h = _bn_update(agg0_ref[...], agg1_ref[...], hlp_ref[...], hp_ref[...],
                   deg_ref[...], dis_ref[...], root_ref[...], gamma_ref[...],
                   beta_ref[...], do_relu=False)
    gids = lax.broadcasted_iota(jnp.int32, (1, G), 1)
    m = (batch_ref[...] == gids).astype(_f32)
    hg_sum = lax.dot_general(m, h, (((0,), (0,)), ((), ())),
                             preferred_element_type=_f32)
    counts = lax.dot_general(m, jnp.ones((N, 1), _f32),
                             (((0,), (0,)), ((), ())),
                             preferred_element_type=_f32)
    hg = hg_sum / jnp.maximum(counts, 1.0)
    o_ref[...] = jnp.dot(hg, wp_ref[...], preferred_element_type=_f32) \
        + bp_ref[...]


def _final_call(agg0, agg1, hl_prev, h_prev, deg, dis, root, gamma, beta,
                batch, wpt, bp):
    out_dim = wpt.shape[1]
    return pl.pallas_call(
        _final_body,
        out_shape=jax.ShapeDtypeStruct((G, out_dim), _f32),
    )(agg0, agg1, hl_prev, h_prev, deg, dis, root, gamma, beta, batch, wpt, bp)


def kernel(x, edge_index, edge_attr, batch, params):
    row = edge_index[0]
    col = edge_index[1]

    deg_p, dis_p, dis_row = _prep_call(row)
    deg = deg_p[:N].reshape(N, 1)
    dis = dis_p[:N].reshape(N, 1)

    hl0row = params['node_emb'][0] @ params['W_lin'][0].T + params['b_lin'][0]
    bias = params['b_edge'].at[0].add(hl0row)
    w7 = jnp.transpose(params['W_edge'], (0, 2, 1))
    ee2 = _ee_call(edge_attr, dis_row.reshape(E // 2560, 20, 128), w7,
                   bias[:, None, :])

    h, hl, hl2 = _node0_call(params['node_emb'], dis,
                             params['W_lin'][0].T, params['b_lin'][0][None])

    for l in range(1, L):
        agg0, agg1 = _agg_call(l - 1, l - 1 > 0, hl2, ee2, row, col)
        h, hl, hl2 = _node_call(
            l - 1 < L - 1, agg0, agg1, hl, h, deg, dis,
            params['root'][l - 1], params['gamma'][l - 1][None],
            params['beta'][l - 1][None],
            params['W_lin'][l].T, params['b_lin'][l][None])

    agg0, agg1 = _agg_call(L - 1, True, hl2, ee2, row, col)
    return _final_call(agg0, agg1, hl, h, deg, dis,
                       params['root'][L - 1], params['gamma'][L - 1][None],
                       params['beta'][L - 1][None],
                       batch.reshape(N, 1), params['W_pred'].T,
                       params['b_pred'][None])
